# Initial kernel scaffold; baseline (speedup 1.0000x reference)
import jax
import jax.numpy as jnp
from jax.experimental import pallas as pl


def kernel(hidden_states, weight):
    raise NotImplementedError("write your pallas kernel here")


# speedup vs baseline: 1.0679x; 1.0679x over previous
import functools

import jax
import jax.numpy as jnp
from jax.experimental import pallas as pl

EMBED = 2048
EXPERTS = 64
K = 8
BLOCK = 512


def _body(x_ref, w_ref, idx_ref, wgt_ref):
    x = x_ref[...]
    w = w_ref[...]
    logits = jax.lax.dot_general(
        x, w, (((1,), (1,)), ((), ())), preferred_element_type=jnp.float32
    )
    m = jnp.max(logits, axis=1, keepdims=True)
    e = jnp.exp(logits - m)
    s = jnp.sum(e, axis=1, keepdims=True)
    scores = e / s
    iota = jax.lax.broadcasted_iota(jnp.int32, scores.shape, 1)
    vals, idxs = [], []
    sc = scores
    for _ in range(K):
        mj = jnp.max(sc, axis=1, keepdims=True)
        ij = jnp.min(jnp.where(sc == mj, iota, EXPERTS), axis=1, keepdims=True)
        vals.append(mj)
        idxs.append(ij)
        sc = jnp.where(iota == ij, -1.0, sc)
    wgt_ref[...] = jnp.concatenate(vals, axis=1)
    idx_ref[...] = jnp.concatenate(idxs, axis=1)


@jax.jit
def kernel(hidden_states, weight):
    x = hidden_states.reshape(-1, EMBED)
    n = x.shape[0]
    grid = n // BLOCK
    idx, wgt = pl.pallas_call(
        _body,
        grid=(grid,),
        in_specs=[
            pl.BlockSpec((BLOCK, EMBED), lambda i: (i, 0)),
            pl.BlockSpec((EXPERTS, EMBED), lambda i: (0, 0)),
        ],
        out_specs=[
            pl.BlockSpec((BLOCK, K), lambda i: (i, 0)),
            pl.BlockSpec((BLOCK, K), lambda i: (i, 0)),
        ],
        out_shape=[
            jax.ShapeDtypeStruct((n, K), jnp.int32),
            jax.ShapeDtypeStruct((n, K), jnp.float32),
        ],
    )(x, weight)
    return (idx, wgt)


# speedup vs baseline: 1.5868x; 1.4859x over previous
import jax
import jax.numpy as jnp
from jax.experimental import pallas as pl

EMBED = 2048
EXPERTS = 64
K = 8
BLOCK = 512


def _body(x_ref, w_ref, idx_ref, wgt_ref):
    x = x_ref[...]
    w = w_ref[...]
    logits = jax.lax.dot_general(
        x, w, (((1,), (1,)), ((), ())), preferred_element_type=jnp.float32
    )
    lt = logits.T
    m = jnp.max(lt, axis=0, keepdims=True)
    e = jnp.exp(lt - m)
    s = jnp.sum(e, axis=0, keepdims=True)
    sc = e / s
    iota = jax.lax.broadcasted_iota(jnp.int32, sc.shape, 0).astype(jnp.float32)
    vals, idxs = [], []
    for _ in range(K):
        mj = jnp.max(sc, axis=0, keepdims=True)
        hit = sc == mj
        ij = jnp.min(jnp.where(hit, iota, float(EXPERTS)), axis=0, keepdims=True)
        vals.append(mj)
        idxs.append(ij)
        sc = jnp.where(iota == ij, -1.0, sc)
    wgt_ref[...] = jnp.concatenate(vals, axis=0).T
    idx_ref[...] = jnp.concatenate(idxs, axis=0).T.astype(jnp.int32)


@jax.jit
def kernel(hidden_states, weight):
    x = hidden_states.reshape(-1, EMBED)
    n = x.shape[0]
    grid = n // BLOCK
    idx, wgt = pl.pallas_call(
        _body,
        grid=(grid,),
        in_specs=[
            pl.BlockSpec((BLOCK, EMBED), lambda i: (i, 0)),
            pl.BlockSpec((EXPERTS, EMBED), lambda i: (0, 0)),
        ],
        out_specs=[
            pl.BlockSpec((BLOCK, K), lambda i: (i, 0)),
            pl.BlockSpec((BLOCK, K), lambda i: (i, 0)),
        ],
        out_shape=[
            jax.ShapeDtypeStruct((n, K), jnp.int32),
            jax.ShapeDtypeStruct((n, K), jnp.float32),
        ],
    )(x, weight)
    return (idx, wgt)


# speedup vs baseline: 1.8675x; 1.1768x over previous
import jax
import jax.numpy as jnp
from jax.experimental import pallas as pl

EMBED = 2048
EXPERTS = 64
K = 8
BLOCK = 1024


def _body(x_ref, w_ref, idx_ref, wgt_ref):
    x = x_ref[...]
    w = w_ref[...]
    logits = jax.lax.dot_general(
        x, w, (((1,), (1,)), ((), ())), preferred_element_type=jnp.float32
    )
    lt = logits.T
    m = jnp.max(lt, axis=0, keepdims=True)
    e = jnp.exp(lt - m)
    s = jnp.sum(e, axis=0, keepdims=True)
    sc = e / s
    iota = jax.lax.broadcasted_iota(jnp.int32, sc.shape, 0).astype(jnp.float32)
    vals, idxs = [], []
    for _ in range(K):
        mj = jnp.max(sc, axis=0, keepdims=True)
        hit = sc == mj
        ij = jnp.min(jnp.where(hit, iota, float(EXPERTS)), axis=0, keepdims=True)
        vals.append(mj)
        idxs.append(ij)
        sc = jnp.where(iota == ij, -1.0, sc)
    wgt_ref[...] = jnp.concatenate(vals, axis=0).T
    idx_ref[...] = jnp.concatenate(idxs, axis=0).T.astype(jnp.int32)


@jax.jit
def kernel(hidden_states, weight):
    x = hidden_states.reshape(-1, EMBED)
    n = x.shape[0]
    grid = n // BLOCK
    idx, wgt = pl.pallas_call(
        _body,
        grid=(grid,),
        in_specs=[
            pl.BlockSpec((BLOCK, EMBED), lambda i: (i, 0)),
            pl.BlockSpec((EXPERTS, EMBED), lambda i: (0, 0)),
        ],
        out_specs=[
            pl.BlockSpec((BLOCK, K), lambda i: (i, 0)),
            pl.BlockSpec((BLOCK, K), lambda i: (i, 0)),
        ],
        out_shape=[
            jax.ShapeDtypeStruct((n, K), jnp.int32),
            jax.ShapeDtypeStruct((n, K), jnp.float32),
        ],
    )(x, weight)
    return (idx, wgt)


# speedup vs baseline: 1.9221x; 1.0293x over previous
import jax
import jax.numpy as jnp
from jax.experimental import pallas as pl

EMBED = 2048
EXPERTS = 64
K = 8
BLOCK = 2048


def _body(x_ref, w_ref, idx_ref, wgt_ref):
    x = x_ref[...]
    w = w_ref[...]
    logits = jax.lax.dot_general(
        x, w, (((1,), (1,)), ((), ())), preferred_element_type=jnp.float32
    )
    lt = logits.T
    m = jnp.max(lt, axis=0, keepdims=True)
    e = jnp.exp(lt - m)
    s = jnp.sum(e, axis=0, keepdims=True)
    sc = e / s
    iota = jax.lax.broadcasted_iota(jnp.int32, sc.shape, 0).astype(jnp.float32)
    vals, idxs = [], []
    for _ in range(K):
        mj = jnp.max(sc, axis=0, keepdims=True)
        hit = sc == mj
        ij = jnp.min(jnp.where(hit, iota, float(EXPERTS)), axis=0, keepdims=True)
        vals.append(mj)
        idxs.append(ij)
        sc = jnp.where(iota == ij, -1.0, sc)
    wgt_ref[...] = jnp.concatenate(vals, axis=0).T
    idx_ref[...] = jnp.concatenate(idxs, axis=0).T.astype(jnp.int32)


@jax.jit
def kernel(hidden_states, weight):
    x = hidden_states.reshape(-1, EMBED)
    n = x.shape[0]
    grid = n // BLOCK
    idx, wgt = pl.pallas_call(
        _body,
        grid=(grid,),
        in_specs=[
            pl.BlockSpec((BLOCK, EMBED), lambda i: (i, 0)),
            pl.BlockSpec((EXPERTS, EMBED), lambda i: (0, 0)),
        ],
        out_specs=[
            pl.BlockSpec((BLOCK, K), lambda i: (i, 0)),
            pl.BlockSpec((BLOCK, K), lambda i: (i, 0)),
        ],
        out_shape=[
            jax.ShapeDtypeStruct((n, K), jnp.int32),
            jax.ShapeDtypeStruct((n, K), jnp.float32),
        ],
    )(x, weight)
    return (idx, wgt)


# speedup vs baseline: 1.9244x; 1.0012x over previous
import jax
import jax.numpy as jnp
from jax.experimental import pallas as pl

EMBED = 2048
HALF = EMBED // 2
EXPERTS = 64
K = 8
BLOCK = 2048


def _body(x1_ref, x2_ref, w_ref, idx_ref, wgt_ref):
    w = w_ref[...]
    l1 = jax.lax.dot_general(
        x1_ref[...], w[:, :HALF], (((1,), (1,)), ((), ())),
        preferred_element_type=jnp.float32,
    )
    l2 = jax.lax.dot_general(
        x2_ref[...], w[:, HALF:], (((1,), (1,)), ((), ())),
        preferred_element_type=jnp.float32,
    )
    logits = l1 + l2
    lt = logits.T
    m = jnp.max(lt, axis=0, keepdims=True)
    e = jnp.exp(lt - m)
    s = jnp.sum(e, axis=0, keepdims=True)
    sc = e / s
    iota = jax.lax.broadcasted_iota(jnp.int32, sc.shape, 0).astype(jnp.float32)
    vals, idxs = [], []
    for _ in range(K):
        mj = jnp.max(sc, axis=0, keepdims=True)
        hit = sc == mj
        ij = jnp.min(jnp.where(hit, iota, float(EXPERTS)), axis=0, keepdims=True)
        vals.append(mj)
        idxs.append(ij)
        sc = jnp.where(iota == ij, -1.0, sc)
    wgt_ref[...] = jnp.concatenate(vals, axis=0).T
    idx_ref[...] = jnp.concatenate(idxs, axis=0).T.astype(jnp.int32)


@jax.jit
def kernel(hidden_states, weight):
    x = hidden_states.reshape(-1, EMBED)
    n = x.shape[0]
    grid = n // BLOCK
    idx, wgt = pl.pallas_call(
        _body,
        grid=(grid,),
        in_specs=[
            pl.BlockSpec((BLOCK, HALF), lambda i: (i, 0)),
            pl.BlockSpec((BLOCK, HALF), lambda i: (i, 1)),
            pl.BlockSpec((EXPERTS, EMBED), lambda i: (0, 0)),
        ],
        out_specs=[
            pl.BlockSpec((BLOCK, K), lambda i: (i, 0)),
            pl.BlockSpec((BLOCK, K), lambda i: (i, 0)),
        ],
        out_shape=[
            jax.ShapeDtypeStruct((n, K), jnp.int32),
            jax.ShapeDtypeStruct((n, K), jnp.float32),
        ],
    )(x, x, weight)
    return (idx, wgt)
